# baseline (speedup 1.0000x reference)
import functools
import math

import jax
import jax.numpy as jnp
from jax import lax
from jax.experimental import pallas as pl
from jax.experimental.pallas import tpu as pltpu
from jax.experimental.pallas import tpu_sc as plsc

B = 100
NP = 100
EG = 3200
N = B * NP
E = B * EG
FDIM = 128
H = 128
C = 10
K1 = math.ceil(0.5 * NP)
K2 = math.ceil(0.5 * K1)
K3 = math.ceil(0.5 * K2)

G = 5
SLOT = 128
R = G * SLOT

_NEG_INF = float("-inf")

_SC_INFO = plsc.get_sparse_core_info()
_NC = _SC_INFO.num_cores
_NS = _SC_INFO.num_subcores
_NW = _NC * _NS
_GPW = -(-B // _NW)
_LANES = 16


def _sc_count_body(src_hbm, dst_hbm, out_hbm, src_v, dst_v, cnt_v):
    wid = lax.axis_index("s") * _NC + lax.axis_index("c")
    ones = jnp.ones((_LANES,), jnp.float32)
    for t in range(_GPW):
        gid = wid * _GPW + t

        @pl.when(gid < B)
        def _():
            pltpu.sync_copy(src_hbm.at[gid], src_v)
            pltpu.sync_copy(dst_hbm.at[gid], dst_v)

            def zero_step(j, _):
                base = pl.multiple_of(j * _LANES, 8)
                cnt_v[pl.ds(base, _LANES)] = jnp.zeros((_LANES,), jnp.float32)
                return 0

            lax.fori_loop(0, NP * NP // _LANES, zero_step, 0)
            off = gid * NP

            def edge_step(j, _):
                base = pl.multiple_of(j * _LANES, 8)
                s = src_v[pl.ds(base, _LANES)] - off
                d = dst_v[pl.ds(base, _LANES)] - off
                plsc.addupdate_scatter(cnt_v, [d * NP + s], ones)
                return 0

            lax.fori_loop(0, EG // _LANES, edge_step, 0)
            pltpu.sync_copy(cnt_v, out_hbm.at[gid])

    return None


@functools.partial(
    pl.kernel,
    mesh=plsc.VectorSubcoreMesh(core_axis_name="c", subcore_axis_name="s"),
    out_type=jax.ShapeDtypeStruct((B, NP * NP), jnp.float32),
    compiler_params=pltpu.CompilerParams(needs_layout_passes=False),
    scratch_types=[
        pltpu.VMEM((EG,), jnp.int32),
        pltpu.VMEM((EG,), jnp.int32),
        pltpu.VMEM((NP * NP,), jnp.float32),
    ],
)
def _sc_count(src_hbm, dst_hbm, out_hbm, src_v, dst_v, cnt_v):
    _sc_count_body(src_hbm, dst_hbm, out_hbm, src_v, dst_v, cnt_v)


def _sag_body(x_ref, cnt_ref, W1, b1, W2, b2, W3, b3, Wp1, bp1, Wp2, bp2,
              fW1, fb1, fW2, fb2, fW3, fb3, out_ref, cbig_ref, hp_ref):
    f32 = jnp.float32

    @pl.when(pl.program_id(0) == 0)
    def _():
        cbig_ref[...] = jnp.zeros((R, R), f32)
        hp_ref[...] = jnp.zeros((R, FDIM), f32)

    for g in range(G):
        cbig_ref[pl.ds(g * SLOT, NP), pl.ds(g * SLOT, NP)] = cnt_ref[0, g]
        hp_ref[pl.ds(g * SLOT, NP), :] = x_ref[0, g]
    cbig = cbig_ref[...]

    si = lax.broadcasted_iota(jnp.int32, (SLOT, SLOT), 0)
    sj = lax.broadcasted_iota(jnp.int32, (SLOT, SLOT), 1)
    tie2 = (sj < si).reshape(1, SLOT, SLOT)

    riota = lax.broadcasted_iota(jnp.int32, (R, 1), 0)
    m = ((riota % SLOT) < NP).astype(f32)
    h = hp_ref[...]
    sel = ((lax.broadcasted_iota(jnp.int32, (G, R), 1) >> 7)
           == lax.broadcasted_iota(jnp.int32, (G, R), 0)).astype(f32)

    stages = ((W1, b1, Wp1, bp1, K1), (W2, b2, Wp2, bp2, K2),
              (W3, b3, Wp2, bp2, K3))
    z = jnp.zeros((G, 3 * H), f32)
    for (W, b, Wp, bp, k) in stages:
        xw = jnp.dot(h, W[...], preferred_element_type=f32)
        deg = m * jnp.dot(cbig, m, preferred_element_type=f32) + m
        dinv = jnp.where(deg > 0, lax.rsqrt(jnp.maximum(deg, 1e-12)), 0.0)
        dm = dinv * m
        A = cbig * dm * jnp.transpose(dm)
        self_w = dinv * dinv * m
        agg = jnp.dot(A, xw, preferred_element_type=f32) + self_w * xw
        h = jnp.maximum((agg + b[...]) * m, 0.0)
        xws = jnp.dot(h, Wp[...], preferred_element_type=f32)
        sc = (jnp.dot(A, xws, preferred_element_type=f32)
              + self_w * xws + bp[...]) * m
        scm = jnp.where(m > 0, sc, _NEG_INF)
        scm3 = scm.reshape(G, SLOT, 1)
        scr3 = jnp.transpose(scm3, (0, 2, 1))
        beats = (scr3 > scm3) | (tie2 & (scr3 == scm3))
        rank = jnp.sum(beats.astype(f32), axis=2, keepdims=True)
        mn = (rank < k).astype(f32).reshape(R, 1)
        h = h * jnp.tanh(sc) * mn
        ssum = jnp.dot(sel, h, preferred_element_type=f32)
        hv = h.reshape(G, SLOT, H)
        mv = (mn > 0).reshape(G, SLOT, 1)
        mx = jnp.max(jnp.where(mv, hv, _NEG_INF), axis=1)
        z = z + jnp.concatenate([ssum / k, mx, ssum], axis=1)
        m = mn
    z = jnp.maximum(jnp.dot(z, fW1[...], preferred_element_type=f32)
                    + fb1[...], 0.0)
    z = jnp.maximum(jnp.dot(z, fW2[...], preferred_element_type=f32)
                    + fb2[...], 0.0)
    z = jnp.dot(z, fW3[...], preferred_element_type=f32) + fb3[...]
    z = z - jnp.max(z, axis=1, keepdims=True)
    z = z - jnp.log(jnp.sum(jnp.exp(z), axis=1, keepdims=True))
    out_ref[0] = z


def _full(shape):
    return pl.BlockSpec(shape, lambda i: (0,) * len(shape))


@jax.jit
def kernel(x, W1, b1, W2, b2, W3, b3, Wp1, bp1, Wp2, bp2,
           fW1, fb1, fW2, fb2, fW3, fb3, edge_index, batch):
    e32 = edge_index.astype(jnp.int32)
    cnt = _sc_count(e32[0].reshape(B, EG), e32[1].reshape(B, EG))
    cnt4 = cnt.reshape(B // G, G, NP, NP)
    x4 = x.reshape(B // G, G, NP, FDIM)
    args = (x4, cnt4,
            W1, b1.reshape(1, H), W2, b2.reshape(1, H), W3, b3.reshape(1, H),
            Wp1, bp1.reshape(1, 1), Wp2, bp2.reshape(1, 1),
            fW1, fb1.reshape(1, H), fW2, fb2.reshape(1, H // 2),
            fW3, fb3.reshape(1, C))
    in_specs = [
        pl.BlockSpec((1, G, NP, FDIM), lambda i: (i, 0, 0, 0)),
        pl.BlockSpec((1, G, NP, NP), lambda i: (i, 0, 0, 0)),
    ] + [_full(a.shape) for a in args[2:]]
    out = pl.pallas_call(
        _sag_body,
        grid=(B // G,),
        in_specs=in_specs,
        out_specs=pl.BlockSpec((1, G, C), lambda i: (i, 0, 0)),
        out_shape=jax.ShapeDtypeStruct((B // G, G, C), jnp.float32),
        scratch_shapes=[pltpu.VMEM((R, R), jnp.float32),
                        pltpu.VMEM((R, FDIM), jnp.float32)],
    )(*args)
    return out.reshape(B, C)


# speedup vs baseline: 1.0529x; 1.0529x over previous
import functools
import math

import jax
import jax.numpy as jnp
from jax import lax
from jax.experimental import pallas as pl
from jax.experimental.pallas import tpu as pltpu
from jax.experimental.pallas import tpu_sc as plsc

B = 100
NP = 100
EG = 3200
N = B * NP
E = B * EG
FDIM = 128
H = 128
C = 10
K1 = math.ceil(0.5 * NP)
K2 = math.ceil(0.5 * K1)
K3 = math.ceil(0.5 * K2)

G = 5
SLOT = 128
R = G * SLOT

_NEG_INF = float("-inf")

_SC_INFO = plsc.get_sparse_core_info()
_NC = _SC_INFO.num_cores
_NS = _SC_INFO.num_subcores
_NW = _NC * _NS
_GPW = -(-B // _NW)
_LANES = 16


def _sc_count_body(src_hbm, dst_hbm, out_hbm, src_v, dst_v, cnt_v):
    wid = lax.axis_index("s") * _NC + lax.axis_index("c")
    ones = jnp.ones((_LANES,), jnp.float32)
    for t in range(_GPW):
        gid = wid * _GPW + t

        @pl.when(gid < B)
        def _():
            pltpu.sync_copy(src_hbm.at[gid], src_v)
            pltpu.sync_copy(dst_hbm.at[gid], dst_v)

            def zero_step(j, _):
                base = pl.multiple_of(j * _LANES, 8)
                cnt_v[pl.ds(base, _LANES)] = jnp.zeros((_LANES,), jnp.float32)
                return 0

            lax.fori_loop(0, NP * NP // _LANES, zero_step, 0)
            off = gid * NP

            def edge_step(j, _):
                base = pl.multiple_of(j * _LANES, 8)
                s = src_v[pl.ds(base, _LANES)] - off
                d = dst_v[pl.ds(base, _LANES)] - off
                plsc.addupdate_scatter(cnt_v, [d * NP + s], ones)
                return 0

            lax.fori_loop(0, EG // _LANES, edge_step, 0)
            pltpu.sync_copy(cnt_v, out_hbm.at[gid])

    return None


@functools.partial(
    pl.kernel,
    mesh=plsc.VectorSubcoreMesh(core_axis_name="c", subcore_axis_name="s"),
    out_type=jax.ShapeDtypeStruct((B, NP * NP), jnp.float32),
    compiler_params=pltpu.CompilerParams(needs_layout_passes=False),
    scratch_types=[
        pltpu.VMEM((EG,), jnp.int32),
        pltpu.VMEM((EG,), jnp.int32),
        pltpu.VMEM((NP * NP,), jnp.float32),
    ],
)
def _sc_count(src_hbm, dst_hbm, out_hbm, src_v, dst_v, cnt_v):
    _sc_count_body(src_hbm, dst_hbm, out_hbm, src_v, dst_v, cnt_v)


def _sag_body(x_ref, cnt_ref, W1, b1, W2, b2, W3, b3, Wp1, bp1, Wp2, bp2,
              fW1, fb1, fW2, fb2, fW3, fb3, out_ref, cbig_ref, hp_ref):
    f32 = jnp.float32

    @pl.when(pl.program_id(0) == 0)
    def _():
        cbig_ref[...] = jnp.zeros((R, R), f32)
        hp_ref[...] = jnp.zeros((R, FDIM), f32)

    for g in range(G):
        cbig_ref[pl.ds(g * SLOT, NP), pl.ds(g * SLOT, NP)] = cnt_ref[0, g]
        hp_ref[pl.ds(g * SLOT, NP), :] = x_ref[0, g]
    cbig = cbig_ref[...]

    ii = lax.broadcasted_iota(jnp.int32, (R, R), 0)
    jj = lax.broadcasted_iota(jnp.int32, (R, R), 1)
    tie_ok = ((ii >> 7) == (jj >> 7)) & (jj < ii)
    same_g = (ii >> 7) == (jj >> 7)

    riota = lax.broadcasted_iota(jnp.int32, (R, 1), 0)
    m = ((riota % SLOT) < NP).astype(f32)
    h = hp_ref[...]
    sel = ((lax.broadcasted_iota(jnp.int32, (G, R), 1) >> 7)
           == lax.broadcasted_iota(jnp.int32, (G, R), 0)).astype(f32)

    stages = ((W1, b1, Wp1, bp1, K1), (W2, b2, Wp2, bp2, K2),
              (W3, b3, Wp2, bp2, K3))
    z = jnp.zeros((G, 3 * H), f32)
    for (W, b, Wp, bp, k) in stages:
        xw = jnp.dot(h, W[...], preferred_element_type=f32)
        deg = m * jnp.dot(cbig, m, preferred_element_type=f32) + m
        dinv = jnp.where(deg > 0, lax.rsqrt(jnp.maximum(deg, 1e-12)), 0.0)
        dm = dinv * m
        A = cbig * dm * jnp.transpose(dm)
        self_w = dinv * dinv * m
        agg = jnp.dot(A, xw, preferred_element_type=f32) + self_w * xw
        h = jnp.maximum((agg + b[...]) * m, 0.0)
        xws = jnp.dot(h, Wp[...], preferred_element_type=f32)
        sc = (jnp.dot(A, xws, preferred_element_type=f32)
              + self_w * xws + bp[...]) * m
        scm = jnp.where(m > 0, sc, _NEG_INF)
        scr = jnp.transpose(scm)
        beats = same_g & (scr > scm) | (tie_ok & (scr == scm))
        rank = jnp.sum(beats.astype(f32), axis=1, keepdims=True)
        mn = (rank < k).astype(f32)
        h = h * jnp.tanh(sc) * mn
        ssum = jnp.dot(sel, h, preferred_element_type=f32)
        hv = h.reshape(G, SLOT, H)
        mv = (mn > 0).reshape(G, SLOT, 1)
        mx = jnp.max(jnp.where(mv, hv, _NEG_INF), axis=1)
        z = z + jnp.concatenate([ssum / k, mx, ssum], axis=1)
        m = mn
    z = jnp.maximum(jnp.dot(z, fW1[...], preferred_element_type=f32)
                    + fb1[...], 0.0)
    z = jnp.maximum(jnp.dot(z, fW2[...], preferred_element_type=f32)
                    + fb2[...], 0.0)
    z = jnp.dot(z, fW3[...], preferred_element_type=f32) + fb3[...]
    z = z - jnp.max(z, axis=1, keepdims=True)
    z = z - jnp.log(jnp.sum(jnp.exp(z), axis=1, keepdims=True))
    out_ref[0] = z


def _full(shape):
    return pl.BlockSpec(shape, lambda i: (0,) * len(shape))


@jax.jit
def kernel(x, W1, b1, W2, b2, W3, b3, Wp1, bp1, Wp2, bp2,
           fW1, fb1, fW2, fb2, fW3, fb3, edge_index, batch):
    e32 = edge_index.astype(jnp.int32)
    cnt = _sc_count(e32[0].reshape(B, EG), e32[1].reshape(B, EG))
    cnt4 = cnt.reshape(B // G, G, NP, NP)
    x4 = x.reshape(B // G, G, NP, FDIM)
    args = (x4, cnt4,
            W1, b1.reshape(1, H), W2, b2.reshape(1, H), W3, b3.reshape(1, H),
            Wp1, bp1.reshape(1, 1), Wp2, bp2.reshape(1, 1),
            fW1, fb1.reshape(1, H), fW2, fb2.reshape(1, H // 2),
            fW3, fb3.reshape(1, C))
    in_specs = [
        pl.BlockSpec((1, G, NP, FDIM), lambda i: (i, 0, 0, 0)),
        pl.BlockSpec((1, G, NP, NP), lambda i: (i, 0, 0, 0)),
    ] + [_full(a.shape) for a in args[2:]]
    out = pl.pallas_call(
        _sag_body,
        grid=(B // G,),
        in_specs=in_specs,
        out_specs=pl.BlockSpec((1, G, C), lambda i: (i, 0, 0)),
        out_shape=jax.ShapeDtypeStruct((B // G, G, C), jnp.float32),
        scratch_shapes=[pltpu.VMEM((R, R), jnp.float32),
                        pltpu.VMEM((R, FDIM), jnp.float32)],
    )(*args)
    return out.reshape(B, C)


# speedup vs baseline: 1.0983x; 1.0431x over previous
import functools
import math

import jax
import jax.numpy as jnp
from jax import lax
from jax.experimental import pallas as pl
from jax.experimental.pallas import tpu as pltpu
from jax.experimental.pallas import tpu_sc as plsc

B = 100
NP = 100
EG = 3200
N = B * NP
E = B * EG
FDIM = 128
H = 128
C = 10
K1 = math.ceil(0.5 * NP)
K2 = math.ceil(0.5 * K1)
K3 = math.ceil(0.5 * K2)

G = 5
SLOT = 128
R = G * SLOT

_NEG_INF = float("-inf")

_SC_INFO = plsc.get_sparse_core_info()
_NC = _SC_INFO.num_cores
_NS = _SC_INFO.num_subcores
_NW = _NC * _NS
_GPW = -(-B // _NW)
_LANES = 16


def _sc_count_body(src_hbm, dst_hbm, out_hbm, src_v, dst_v, cnt_v):
    wid = lax.axis_index("s") * _NC + lax.axis_index("c")
    ones = jnp.ones((_LANES,), jnp.float32)
    for t in range(_GPW):
        gid = wid * _GPW + t

        @pl.when(gid < B)
        def _():
            pltpu.sync_copy(src_hbm.at[gid], src_v)
            pltpu.sync_copy(dst_hbm.at[gid], dst_v)

            def zero_step(j, _):
                base = pl.multiple_of(j * _LANES, 8)
                cnt_v[pl.ds(base, _LANES)] = jnp.zeros((_LANES,), jnp.float32)
                return 0

            lax.fori_loop(0, NP * NP // _LANES, zero_step, 0)
            off = gid * NP

            def edge_step(j, _):
                base = pl.multiple_of(j * _LANES, 8)
                s = src_v[pl.ds(base, _LANES)] - off
                d = dst_v[pl.ds(base, _LANES)] - off
                plsc.addupdate_scatter(cnt_v, [d * NP + s], ones)
                return 0

            lax.fori_loop(0, EG // _LANES, edge_step, 0)
            pltpu.sync_copy(cnt_v, out_hbm.at[gid])

    return None


@functools.partial(
    pl.kernel,
    mesh=plsc.VectorSubcoreMesh(core_axis_name="c", subcore_axis_name="s"),
    out_type=jax.ShapeDtypeStruct((B, NP * NP), jnp.float32),
    compiler_params=pltpu.CompilerParams(needs_layout_passes=False),
    scratch_types=[
        pltpu.VMEM((EG,), jnp.int32),
        pltpu.VMEM((EG,), jnp.int32),
        pltpu.VMEM((NP * NP,), jnp.float32),
    ],
)
def _sc_count(src_hbm, dst_hbm, out_hbm, src_v, dst_v, cnt_v):
    _sc_count_body(src_hbm, dst_hbm, out_hbm, src_v, dst_v, cnt_v)


def _rowcast(v):
    return lax.dot_general(jnp.ones((1, 1), jnp.float32), v,
                           (((1,), (1,)), ((), ())),
                           preferred_element_type=jnp.float32)


def _sag_body(x_ref, cnt_ref, W1, b1, W2, b2, W3, b3, Wp1, bp1, Wp2, bp2,
              fW1, fb1, fW2, fb2, fW3, fb3, out_ref, cbig_ref, hp_ref):
    f32 = jnp.float32

    @pl.when(pl.program_id(0) == 0)
    def _():
        cbig_ref[...] = jnp.zeros((R, R), f32)
        hp_ref[...] = jnp.zeros((R, FDIM), f32)

    for g in range(G):
        cbig_ref[pl.ds(g * SLOT, NP), pl.ds(g * SLOT, NP)] = cnt_ref[0, g]
        hp_ref[pl.ds(g * SLOT, NP), :] = x_ref[0, g]
    cbig = cbig_ref[...]

    ii = lax.broadcasted_iota(jnp.int32, (R, R), 0)
    jj = lax.broadcasted_iota(jnp.int32, (R, R), 1)
    tie_ok = ((ii >> 7) == (jj >> 7)) & (jj < ii)
    same_g = (ii >> 7) == (jj >> 7)

    riota = lax.broadcasted_iota(jnp.int32, (R, 1), 0)
    m = ((riota % SLOT) < NP).astype(f32)
    h = hp_ref[...]
    sel = ((lax.broadcasted_iota(jnp.int32, (G, R), 1) >> 7)
           == lax.broadcasted_iota(jnp.int32, (G, R), 0)).astype(f32)

    stages = ((W1, b1, Wp1, bp1, K1), (W2, b2, Wp2, bp2, K2),
              (W3, b3, Wp2, bp2, K3))
    z = jnp.zeros((G, 3 * H), f32)
    for (W, b, Wp, bp, k) in stages:
        xw = jnp.dot(h, W[...], preferred_element_type=f32)
        deg = m * jnp.dot(cbig, m, preferred_element_type=f32) + m
        dinv = jnp.where(deg > 0, lax.rsqrt(jnp.maximum(deg, 1e-12)), 0.0)
        dm = dinv * m
        self_w = dinv * dinv * m
        agg = dm * jnp.dot(cbig, dm * xw, preferred_element_type=f32) \
            + self_w * xw
        h = jnp.maximum((agg + b[...]) * m, 0.0)
        xws = jnp.dot(h, Wp[...], preferred_element_type=f32)
        sc = (dm * jnp.dot(cbig, dm * xws, preferred_element_type=f32)
              + self_w * xws + bp[...]) * m
        scm = jnp.where(m > 0, sc, _NEG_INF)
        scr = _rowcast(scm)
        beats = same_g & (scr > scm) | (tie_ok & (scr == scm))
        rank = jnp.sum(beats.astype(f32), axis=1, keepdims=True)
        mn = (rank < k).astype(f32)
        h = h * jnp.tanh(sc) * mn
        ssum = jnp.dot(sel, h, preferred_element_type=f32)
        hv = h.reshape(G, SLOT, H)
        mv = (mn > 0).reshape(G, SLOT, 1)
        mx = jnp.max(jnp.where(mv, hv, _NEG_INF), axis=1)
        z = z + jnp.concatenate([ssum / k, mx, ssum], axis=1)
        m = mn
    z = jnp.maximum(jnp.dot(z, fW1[...], preferred_element_type=f32)
                    + fb1[...], 0.0)
    z = jnp.maximum(jnp.dot(z, fW2[...], preferred_element_type=f32)
                    + fb2[...], 0.0)
    z = jnp.dot(z, fW3[...], preferred_element_type=f32) + fb3[...]
    z = z - jnp.max(z, axis=1, keepdims=True)
    z = z - jnp.log(jnp.sum(jnp.exp(z), axis=1, keepdims=True))
    out_ref[0] = z


def _full(shape):
    return pl.BlockSpec(shape, lambda i: (0,) * len(shape))


@jax.jit
def kernel(x, W1, b1, W2, b2, W3, b3, Wp1, bp1, Wp2, bp2,
           fW1, fb1, fW2, fb2, fW3, fb3, edge_index, batch):
    e32 = edge_index.astype(jnp.int32)
    cnt = _sc_count(e32[0].reshape(B, EG), e32[1].reshape(B, EG))
    cnt4 = cnt.reshape(B // G, G, NP, NP)
    x4 = x.reshape(B // G, G, NP, FDIM)
    args = (x4, cnt4,
            W1, b1.reshape(1, H), W2, b2.reshape(1, H), W3, b3.reshape(1, H),
            Wp1, bp1.reshape(1, 1), Wp2, bp2.reshape(1, 1),
            fW1, fb1.reshape(1, H), fW2, fb2.reshape(1, H // 2),
            fW3, fb3.reshape(1, C))
    in_specs = [
        pl.BlockSpec((1, G, NP, FDIM), lambda i: (i, 0, 0, 0)),
        pl.BlockSpec((1, G, NP, NP), lambda i: (i, 0, 0, 0)),
    ] + [_full(a.shape) for a in args[2:]]
    out = pl.pallas_call(
        _sag_body,
        grid=(B // G,),
        in_specs=in_specs,
        out_specs=pl.BlockSpec((1, G, C), lambda i: (i, 0, 0)),
        out_shape=jax.ShapeDtypeStruct((B // G, G, C), jnp.float32),
        scratch_shapes=[pltpu.VMEM((R, R), jnp.float32),
                        pltpu.VMEM((R, FDIM), jnp.float32)],
    )(*args)
    return out.reshape(B, C)


# speedup vs baseline: 1.1650x; 1.0607x over previous
import functools
import math

import jax
import jax.numpy as jnp
from jax import lax
from jax.experimental import pallas as pl
from jax.experimental.pallas import tpu as pltpu
from jax.experimental.pallas import tpu_sc as plsc

B = 100
NP = 100
EG = 3200
N = B * NP
E = B * EG
FDIM = 128
H = 128
C = 10
K1 = math.ceil(0.5 * NP)
K2 = math.ceil(0.5 * K1)
K3 = math.ceil(0.5 * K2)

G = 5
SLOT = 128
R = G * SLOT

_NEG_INF = float("-inf")

_SC_INFO = plsc.get_sparse_core_info()
_NC = _SC_INFO.num_cores
_NS = _SC_INFO.num_subcores
_NW = _NC * _NS
_GPW = -(-B // _NW)
_LANES = 16


def _sc_count_body(src_hbm, dst_hbm, out_hbm, src_v, dst_v, cnt_v):
    wid = lax.axis_index("s") * _NC + lax.axis_index("c")
    ones = jnp.ones((_LANES,), jnp.float32)
    for t in range(_GPW):
        gid = wid * _GPW + t

        @pl.when(gid < B)
        def _():
            pltpu.sync_copy(src_hbm.at[gid], src_v)
            pltpu.sync_copy(dst_hbm.at[gid], dst_v)

            def zero_step(j, _):
                base = pl.multiple_of(j * _LANES, 8)
                cnt_v[pl.ds(base, _LANES)] = jnp.zeros((_LANES,), jnp.float32)
                return 0

            lax.fori_loop(0, NP * NP // _LANES, zero_step, 0)
            off = gid * NP

            def edge_step(j, _):
                base = pl.multiple_of(j * _LANES, 8)
                s = src_v[pl.ds(base, _LANES)] - off
                d = dst_v[pl.ds(base, _LANES)] - off
                plsc.addupdate_scatter(cnt_v, [d * NP + s], ones)
                return 0

            lax.fori_loop(0, EG // _LANES, edge_step, 0)
            pltpu.sync_copy(cnt_v, out_hbm.at[gid])

    return None


@functools.partial(
    pl.kernel,
    mesh=plsc.VectorSubcoreMesh(core_axis_name="c", subcore_axis_name="s"),
    out_type=jax.ShapeDtypeStruct((B, NP * NP), jnp.float32),
    compiler_params=pltpu.CompilerParams(needs_layout_passes=False),
    scratch_types=[
        pltpu.VMEM((EG,), jnp.int32),
        pltpu.VMEM((EG,), jnp.int32),
        pltpu.VMEM((NP * NP,), jnp.float32),
    ],
)
def _sc_count(src_hbm, dst_hbm, out_hbm, src_v, dst_v, cnt_v):
    _sc_count_body(src_hbm, dst_hbm, out_hbm, src_v, dst_v, cnt_v)


def _rowcast(v):
    return lax.dot_general(jnp.ones((1, 1), jnp.float32), v,
                           (((1,), (1,)), ((), ())),
                           preferred_element_type=jnp.float32)


def _sag_body(x_ref, cnt_ref, W1, b1, W2, b2, W3, b3, Wp1, bp1, Wp2, bp2,
              fW1, fb1, fW2, fb2, fW3, fb3, out_ref, cbig_ref, hp_ref):
    f32 = jnp.float32

    @pl.when(pl.program_id(0) == 0)
    def _():
        cbig_ref[...] = jnp.zeros((R, R), f32)
        hp_ref[...] = jnp.zeros((R, FDIM), f32)

    for g in range(G):
        cbig_ref[pl.ds(g * SLOT, NP), pl.ds(g * SLOT, NP)] = cnt_ref[0, g]
        hp_ref[pl.ds(g * SLOT, NP), :] = x_ref[0, g]
    cbig = cbig_ref[...]

    ii = lax.broadcasted_iota(jnp.int32, (R, R), 0)
    jj = lax.broadcasted_iota(jnp.int32, (R, R), 1)
    tie_ok = ((ii >> 7) == (jj >> 7)) & (jj < ii)
    same_g = (ii >> 7) == (jj >> 7)

    riota = lax.broadcasted_iota(jnp.int32, (R, 1), 0)
    m = ((riota % SLOT) < NP).astype(f32)
    h = hp_ref[...]
    sel = ((lax.broadcasted_iota(jnp.int32, (G, R), 1) >> 7)
           == lax.broadcasted_iota(jnp.int32, (G, R), 0)).astype(f32)

    stages = ((W1, b1, Wp1, bp1, K1), (W2, b2, Wp2, bp2, K2),
              (W3, b3, Wp2, bp2, K3))
    z = jnp.zeros((G, 3 * H), f32)
    for (W, b, Wp, bp, k) in stages:
        xw = jnp.dot(h, W[...], preferred_element_type=f32)
        deg = m * jnp.sum(cbig * _rowcast(m), axis=1, keepdims=True) + m
        dinv = jnp.where(deg > 0, lax.rsqrt(jnp.maximum(deg, 1e-12)), 0.0)
        dm = dinv * m
        self_w = dinv * dinv * m
        agg = dm * jnp.dot(cbig, dm * xw, preferred_element_type=f32) \
            + self_w * xw
        h = jnp.maximum((agg + b[...]) * m, 0.0)
        xws = jnp.dot(h, Wp[...], preferred_element_type=f32)
        sc = (dm * jnp.sum(cbig * _rowcast(dm * xws), axis=1, keepdims=True)
              + self_w * xws + bp[...]) * m
        scm = jnp.where(m > 0, sc, _NEG_INF)
        scr = _rowcast(scm)
        beats = same_g & (scr > scm) | (tie_ok & (scr == scm))
        rank = jnp.sum(beats.astype(f32), axis=1, keepdims=True)
        mn = (rank < k).astype(f32)
        h = h * jnp.tanh(sc) * mn
        ssum = jnp.dot(sel, h, preferred_element_type=f32)
        hv = h.reshape(G, SLOT, H)
        mv = (mn > 0).reshape(G, SLOT, 1)
        mx = jnp.max(jnp.where(mv, hv, _NEG_INF), axis=1)
        z = z + jnp.concatenate([ssum / k, mx, ssum], axis=1)
        m = mn
    z = jnp.maximum(jnp.dot(z, fW1[...], preferred_element_type=f32)
                    + fb1[...], 0.0)
    z = jnp.maximum(jnp.dot(z, fW2[...], preferred_element_type=f32)
                    + fb2[...], 0.0)
    z = jnp.dot(z, fW3[...], preferred_element_type=f32) + fb3[...]
    z = z - jnp.max(z, axis=1, keepdims=True)
    z = z - jnp.log(jnp.sum(jnp.exp(z), axis=1, keepdims=True))
    out_ref[0] = z


def _full(shape):
    return pl.BlockSpec(shape, lambda i: (0,) * len(shape))


@jax.jit
def kernel(x, W1, b1, W2, b2, W3, b3, Wp1, bp1, Wp2, bp2,
           fW1, fb1, fW2, fb2, fW3, fb3, edge_index, batch):
    e32 = edge_index.astype(jnp.int32)
    cnt = _sc_count(e32[0].reshape(B, EG), e32[1].reshape(B, EG))
    cnt4 = cnt.reshape(B // G, G, NP, NP)
    x4 = x.reshape(B // G, G, NP, FDIM)
    args = (x4, cnt4,
            W1, b1.reshape(1, H), W2, b2.reshape(1, H), W3, b3.reshape(1, H),
            Wp1, bp1.reshape(1, 1), Wp2, bp2.reshape(1, 1),
            fW1, fb1.reshape(1, H), fW2, fb2.reshape(1, H // 2),
            fW3, fb3.reshape(1, C))
    in_specs = [
        pl.BlockSpec((1, G, NP, FDIM), lambda i: (i, 0, 0, 0)),
        pl.BlockSpec((1, G, NP, NP), lambda i: (i, 0, 0, 0)),
    ] + [_full(a.shape) for a in args[2:]]
    out = pl.pallas_call(
        _sag_body,
        grid=(B // G,),
        in_specs=in_specs,
        out_specs=pl.BlockSpec((1, G, C), lambda i: (i, 0, 0)),
        out_shape=jax.ShapeDtypeStruct((B // G, G, C), jnp.float32),
        scratch_shapes=[pltpu.VMEM((R, R), jnp.float32),
                        pltpu.VMEM((R, FDIM), jnp.float32)],
    )(*args)
    return out.reshape(B, C)
